# Initial kernel scaffold; baseline (speedup 1.0000x reference)
import jax
import jax.numpy as jnp
from jax.experimental import pallas as pl


def kernel(x, edge_index, pos, W_node, b_node, W_e1, b_e1, W_e2, b_e2, W_r1, b_r1, W_r2, b_r2):
    raise NotImplementedError("write your pallas kernel here")


# speedup vs baseline: 2.6118x; 2.6118x over previous
import functools

import jax
import jax.numpy as jnp
from jax import lax
from jax.experimental import pallas as pl
from jax.experimental.pallas import tpu as pltpu
from jax.experimental.pallas import tpu_sc as plsc

F = 128
CHUNK = 128
LANES = 16
TC_BLK = 1024
EDGE_BLK = 512


def _silu(v):
    return v * jax.nn.sigmoid(v)


def _matmul_body(x_ref, w_ref, o_ref):
    o_ref[...] = jnp.dot(x_ref[...], w_ref[...],
                         preferred_element_type=jnp.float32)


def _node_premul(x_pad, w):
    n_pad = x_pad.shape[0]
    return pl.pallas_call(
        _matmul_body,
        grid=(n_pad // TC_BLK,),
        in_specs=[pl.BlockSpec((TC_BLK, F), lambda i: (i, 0)),
                  pl.BlockSpec((F, F), lambda i: (0, 0))],
        out_specs=pl.BlockSpec((TC_BLK, F), lambda i: (i, 0)),
        out_shape=jax.ShapeDtypeStruct((n_pad, F), jnp.float32),
    )(x_pad, w)


def _make_gather(nc, ns, cpw, e_pad, n_pad):
    nw = nc * ns
    mesh = plsc.VectorSubcoreMesh(core_axis_name="c", subcore_axis_name="s")

    @functools.partial(
        pl.kernel,
        out_type=(jax.ShapeDtypeStruct((e_pad, F), jnp.float32),
                  jax.ShapeDtypeStruct((e_pad,), jnp.float32)),
        mesh=mesh,
        scratch_types=[
            pltpu.VMEM((n_pad * 4,), jnp.float32),
            pltpu.VMEM((CHUNK,), jnp.int32),
            pltpu.VMEM((CHUNK,), jnp.int32),
            pltpu.VMEM((CHUNK, F), jnp.float32),
            pltpu.VMEM((CHUNK,), jnp.float32),
            pltpu.SemaphoreType.DMA,
        ],
        compiler_params=pltpu.CompilerParams(needs_layout_passes=False),
    )
    def gather_kernel(xe1, src3, dst3, pos4, g_out, d2_out,
                      pos_v, src_v, dst_v, rows_v, d2_v, sem):
        wid = lax.axis_index("s") * nc + lax.axis_index("c")
        pltpu.sync_copy(pos4, pos_v)

        def body(jj, carry):
            j = wid * cpw + jj
            base = j * CHUNK
            pltpu.sync_copy(src3.at[wid, jj], src_v)
            pltpu.sync_copy(dst3.at[wid, jj], dst_v)
            pltpu.async_copy(xe1.at[src_v], rows_v, sem).wait()
            pltpu.sync_copy(rows_v, g_out.at[pl.ds(base, CHUNK)])
            for grp in range(CHUNK // LANES):
                sv = src_v[pl.ds(grp * LANES, LANES)] * 4
                dv = dst_v[pl.ds(grp * LANES, LANES)] * 4
                acc = jnp.zeros((LANES,), jnp.float32)
                for cdim in range(3):
                    ps = plsc.load_gather(pos_v, [sv + cdim])
                    pd = plsc.load_gather(pos_v, [dv + cdim])
                    d = pd - ps
                    acc = acc + d * d
                d2_v[pl.ds(grp * LANES, LANES)] = acc
            pltpu.sync_copy(d2_v, d2_out.at[pl.ds(base, CHUNK)])
            return carry

        lax.fori_loop(0, cpw, body, 0)

    return gather_kernel


def _edge_mlp_body(g_ref, d2_ref, wr1_ref, br1_ref, wr2_ref, br2_ref,
                   we1b_ref, be1_ref, we2_ref, be2_ref, o_ref):
    g = g_ref[...]
    l2 = d2_ref[...] + 1e-12
    l = jnp.sqrt(l2)
    hid = _silu(l[:, None] * wr1_ref[...][None, :] + br1_ref[...][None, :])
    rad = _silu(jnp.dot(hid, wr2_ref[...], preferred_element_type=jnp.float32)
                + br2_ref[...][None, :])
    h = _silu(g + jnp.dot(rad, we1b_ref[...], preferred_element_type=jnp.float32)
              + be1_ref[...][None, :])
    e = _silu(jnp.dot(h, we2_ref[...], preferred_element_type=jnp.float32)
              + be2_ref[...][None, :])
    cf = jnp.clip(1.0 - (l * 0.1) ** 2, 0.0, 1.0) * (l < 10.0).astype(jnp.float32)
    o_ref[...] = e * cf[:, None]


def _edge_mlp(g, d2, wr1v, br1v, wr2p, br2v, we1bp, be1, we2, be2):
    e_pad = g.shape[0]
    vec = pl.BlockSpec((F,), lambda i: (0,))
    mat = pl.BlockSpec((F, F), lambda i: (0, 0))
    return pl.pallas_call(
        _edge_mlp_body,
        grid=(e_pad // EDGE_BLK,),
        in_specs=[pl.BlockSpec((EDGE_BLK, F), lambda i: (i, 0)),
                  pl.BlockSpec((EDGE_BLK,), lambda i: (i,)),
                  vec, vec, mat, vec, mat, vec, mat, vec],
        out_specs=pl.BlockSpec((EDGE_BLK, F), lambda i: (i, 0)),
        out_shape=jax.ShapeDtypeStruct((e_pad, F), jnp.float32),
    )(g, d2, wr1v, br1v, wr2p, br2v, we1bp, be1, we2, be2)


def _make_scatter(nc, ns, cpw, e_pad, n_pad):
    mesh = plsc.VectorSubcoreMesh(core_axis_name="c", subcore_axis_name="s")
    rows_per_sub = n_pad // ns

    @functools.partial(
        pl.kernel,
        out_type=jax.ShapeDtypeStruct((nc, n_pad, F), jnp.float32),
        mesh=mesh,
        scratch_types=[
            pltpu.VMEM((CHUNK,), jnp.int32),
            pltpu.VMEM((CHUNK, F), jnp.float32),
            pltpu.VMEM_SHARED((n_pad, F), jnp.float32),
        ],
        compiler_params=pltpu.CompilerParams(needs_layout_passes=False),
    )
    def scatter_kernel(ef, tgt3, zinit, part_out, idx_v, rows_v, accum):
        c = lax.axis_index("c")
        s = lax.axis_index("s")
        wid = s * nc + c
        pltpu.sync_copy(zinit, accum.at[pl.ds(s * rows_per_sub, rows_per_sub)])
        plsc.subcore_barrier()

        def body(jj, carry):
            j = wid * cpw + jj
            base = j * CHUNK
            pltpu.sync_copy(tgt3.at[wid, jj], idx_v)
            pltpu.sync_copy(ef.at[pl.ds(base, CHUNK)], rows_v)
            pltpu.sync_copy(rows_v, accum.at[idx_v], add=True)
            return carry

        lax.fori_loop(0, cpw, body, 0)
        plsc.subcore_barrier()
        pltpu.sync_copy(accum.at[pl.ds(s * rows_per_sub, rows_per_sub)],
                        part_out.at[c, pl.ds(s * rows_per_sub, rows_per_sub)])

    return scatter_kernel


def _final_body(x_ref, w_ref, b_ref, p0_ref, p1_ref, o_ref):
    o_ref[...] = (jnp.dot(x_ref[...], w_ref[...],
                          preferred_element_type=jnp.float32)
                  + b_ref[...][None, :] + p0_ref[...] + p1_ref[...])


def _final(x_pad, w, b, p0, p1):
    n_pad = x_pad.shape[0]
    blk = pl.BlockSpec((TC_BLK, F), lambda i: (i, 0))
    return pl.pallas_call(
        _final_body,
        grid=(n_pad // TC_BLK,),
        in_specs=[blk,
                  pl.BlockSpec((F, F), lambda i: (0, 0)),
                  pl.BlockSpec((F,), lambda i: (0,)),
                  blk, blk],
        out_specs=blk,
        out_shape=jax.ShapeDtypeStruct((n_pad, F), jnp.float32),
    )(x_pad, w, b, p0, p1)


def kernel(x, edge_index, pos, W_node, b_node, W_e1, b_e1, W_e2, b_e2,
           W_r1, b_r1, W_r2, b_r2):
    N, Fin = x.shape
    E = edge_index.shape[1]
    mesh = plsc.VectorSubcoreMesh(core_axis_name="c", subcore_axis_name="s")
    nc, ns = mesh.num_cores, mesh.num_subcores
    nw = nc * ns
    cpw = -(-E // (nw * CHUNK))
    e_pad = nw * CHUNK * cpw
    n_pad = -(-N // TC_BLK) * TC_BLK

    src = edge_index[0].astype(jnp.int32)
    dst = edge_index[1].astype(jnp.int32)
    src3 = jnp.concatenate(
        [src, jnp.full((e_pad - E,), N, jnp.int32)]).reshape(nw, cpw, CHUNK)
    dst3 = jnp.concatenate(
        [dst, jnp.zeros((e_pad - E,), jnp.int32)]).reshape(nw, cpw, CHUNK)
    tgt3 = dst3

    x_pad = jnp.pad(x, ((0, n_pad - N), (0, 0)))
    pos4 = jnp.full((n_pad, 4), 1e6, jnp.float32)
    pos4 = pos4.at[:N, :3].set(pos).reshape(-1)

    r_hid = W_r1.shape[1]
    r_out = W_r2.shape[1]
    wr1v = jnp.pad(W_r1[0], (0, F - r_hid))
    br1v = jnp.pad(b_r1, (0, F - r_hid))
    wr2p = jnp.zeros((F, F), jnp.float32).at[:r_hid, :r_out].set(W_r2)
    br2v = jnp.pad(b_r2, (0, F - r_out))
    we1bp = jnp.zeros((F, F), jnp.float32).at[:r_out].set(W_e1[Fin:])
    zinit = jnp.zeros((n_pad // ns, F), jnp.float32)

    xe1 = _node_premul(x_pad, W_e1[:Fin])
    g, d2 = _make_gather(nc, ns, cpw, e_pad, n_pad)(xe1, src3, dst3, pos4)
    ef = _edge_mlp(g, d2, wr1v, br1v, wr2p, br2v, we1bp, b_e1, W_e2, b_e2)
    part = _make_scatter(nc, ns, cpw, e_pad, n_pad)(ef, tgt3, zinit)
    out_pad = _final(x_pad, W_node, b_node, part[0], part[1])
    return out_pad[:N]


# speedup vs baseline: 2.9445x; 1.1274x over previous
import functools

import jax
import jax.numpy as jnp
from jax import lax
from jax.experimental import pallas as pl
from jax.experimental.pallas import tpu as pltpu
from jax.experimental.pallas import tpu_sc as plsc

F = 128
CHUNK = 128
LANES = 16
TC_BLK = 1024
EDGE_BLK = 512


def _silu(v):
    return v * jax.nn.sigmoid(v)


def _matmul_body(x_ref, w_ref, o_ref):
    o_ref[...] = jnp.dot(x_ref[...], w_ref[...],
                         preferred_element_type=jnp.float32)


def _node_premul(x_pad, w):
    n_pad = x_pad.shape[0]
    return pl.pallas_call(
        _matmul_body,
        grid=(n_pad // TC_BLK,),
        in_specs=[pl.BlockSpec((TC_BLK, F), lambda i: (i, 0)),
                  pl.BlockSpec((F, F), lambda i: (0, 0))],
        out_specs=pl.BlockSpec((TC_BLK, F), lambda i: (i, 0)),
        out_shape=jax.ShapeDtypeStruct((n_pad, F), jnp.float32),
    )(x_pad, w)


RING = 2


def _make_gather(nc, ns, cpw, e_pad, n_pad):
    nw = nc * ns
    mesh = plsc.VectorSubcoreMesh(core_axis_name="c", subcore_axis_name="s")

    @functools.partial(
        pl.kernel,
        out_type=(jax.ShapeDtypeStruct((e_pad, F), jnp.float32),
                  jax.ShapeDtypeStruct((e_pad // CHUNK, CHUNK), jnp.float32)),
        mesh=mesh,
        scratch_types=[
            pltpu.VMEM((n_pad * 3,), jnp.float32),
            pltpu.VMEM((cpw, CHUNK), jnp.int32),
            pltpu.VMEM((cpw, CHUNK), jnp.int32),
            pltpu.VMEM((RING, CHUNK, F), jnp.float32),
            pltpu.VMEM((cpw, CHUNK), jnp.float32),
        ] + [pltpu.SemaphoreType.DMA] * (2 * RING),
        compiler_params=pltpu.CompilerParams(needs_layout_passes=False),
    )
    def gather_kernel(xe1, src3, dst3, pos3, g_out, d2_out,
                      pos_v, src_all, dst_all, rows_v, d2_all, *sems):
        semg = sems[:RING]
        semw = sems[RING:]
        wid = lax.axis_index("s") * nc + lax.axis_index("c")
        pltpu.sync_copy(pos3, pos_v)
        pltpu.sync_copy(src3.at[wid], src_all)
        pltpu.sync_copy(dst3.at[wid], dst_all)

        def issue_gather(c, b):
            pltpu.async_copy(xe1.at[src_all.at[c]], rows_v.at[b], semg[b])

        for b in range(RING):
            issue_gather(b, b)

        def body(g, carry):
            for b in range(RING):
                c = g * RING + b
                j = wid * cpw + c
                pltpu.make_async_copy(
                    xe1.at[src_all.at[c]], rows_v.at[b], semg[b]).wait()
                out_slice = g_out.at[pl.ds(j * CHUNK, CHUNK)]
                pltpu.async_copy(rows_v.at[b], out_slice, semw[b])
                for grp in range(CHUNK // LANES):
                    sv = src_all[c, pl.ds(grp * LANES, LANES)] * 3
                    dv = dst_all[c, pl.ds(grp * LANES, LANES)] * 3
                    acc = jnp.zeros((LANES,), jnp.float32)
                    for cdim in range(3):
                        ps = plsc.load_gather(pos_v, [sv + cdim])
                        pd = plsc.load_gather(pos_v, [dv + cdim])
                        d = pd - ps
                        acc = acc + d * d
                    d2_all[c, pl.ds(grp * LANES, LANES)] = acc
                pltpu.make_async_copy(rows_v.at[b], out_slice, semw[b]).wait()

                @pl.when(c + RING < cpw)
                def _():
                    issue_gather(c + RING, b)
            return carry

        lax.fori_loop(0, cpw // RING, body, 0)
        pltpu.sync_copy(d2_all, d2_out.at[pl.ds(wid * cpw, cpw)])

    return gather_kernel


def _edge_mlp_body(g_ref, d2_ref, wr1_ref, br1_ref, wr2_ref, br2_ref,
                   we1b_ref, be1_ref, we2_ref, be2_ref, o_ref):
    g = g_ref[...]
    l2 = d2_ref[...] + 1e-12
    l = jnp.sqrt(l2)
    hid = _silu(l[:, None] * wr1_ref[...][None, :] + br1_ref[...][None, :])
    rad = _silu(jnp.dot(hid, wr2_ref[...], preferred_element_type=jnp.float32)
                + br2_ref[...][None, :])
    h = _silu(g + jnp.dot(rad, we1b_ref[...], preferred_element_type=jnp.float32)
              + be1_ref[...][None, :])
    e = _silu(jnp.dot(h, we2_ref[...], preferred_element_type=jnp.float32)
              + be2_ref[...][None, :])
    cf = jnp.clip(1.0 - (l * 0.1) ** 2, 0.0, 1.0) * (l < 10.0).astype(jnp.float32)
    o_ref[...] = e * cf[:, None]


def _edge_mlp(g, d2, wr1v, br1v, wr2p, br2v, we1bp, be1, we2, be2):
    e_pad = g.shape[0]
    vec = pl.BlockSpec((F,), lambda i: (0,))
    mat = pl.BlockSpec((F, F), lambda i: (0, 0))
    return pl.pallas_call(
        _edge_mlp_body,
        grid=(e_pad // EDGE_BLK,),
        in_specs=[pl.BlockSpec((EDGE_BLK, F), lambda i: (i, 0)),
                  pl.BlockSpec((EDGE_BLK,), lambda i: (i,)),
                  vec, vec, mat, vec, mat, vec, mat, vec],
        out_specs=pl.BlockSpec((EDGE_BLK, F), lambda i: (i, 0)),
        out_shape=jax.ShapeDtypeStruct((e_pad, F), jnp.float32),
    )(g, d2, wr1v, br1v, wr2p, br2v, we1bp, be1, we2, be2)


def _make_scatter(nc, ns, cpw, e_pad, n_pad):
    mesh = plsc.VectorSubcoreMesh(core_axis_name="c", subcore_axis_name="s")
    rows_per_sub = n_pad // ns

    @functools.partial(
        pl.kernel,
        out_type=jax.ShapeDtypeStruct((nc, n_pad, F), jnp.float32),
        mesh=mesh,
        scratch_types=[
            pltpu.VMEM((cpw, CHUNK), jnp.int32),
            pltpu.VMEM((RING, CHUNK, F), jnp.float32),
            pltpu.VMEM_SHARED((n_pad, F), jnp.float32),
        ] + [pltpu.SemaphoreType.DMA] * (2 * RING),
        compiler_params=pltpu.CompilerParams(needs_layout_passes=False),
    )
    def scatter_kernel(ef, tgt3, zinit, part_out, tgt_all, rows_v, accum,
                       *sems):
        seml = sems[:RING]
        sema = sems[RING:]
        c_ax = lax.axis_index("c")
        s_ax = lax.axis_index("s")
        wid = s_ax * nc + c_ax
        pltpu.sync_copy(tgt3.at[wid], tgt_all)
        pltpu.sync_copy(zinit, accum.at[pl.ds(s_ax * rows_per_sub,
                                              rows_per_sub)])
        plsc.subcore_barrier()

        def issue_load(c, b):
            j = wid * cpw + c
            pltpu.async_copy(ef.at[pl.ds(j * CHUNK, CHUNK)],
                             rows_v.at[b], seml[b])

        for b in range(RING):
            issue_load(b, b)

        def body(g, carry):
            for b in range(RING):
                c = g * RING + b
                j = wid * cpw + c
                pltpu.make_async_copy(ef.at[pl.ds(j * CHUNK, CHUNK)],
                                      rows_v.at[b], seml[b]).wait()
                pltpu.async_copy(rows_v.at[b], accum.at[tgt_all.at[c]],
                                 sema[b], add=True)
                pltpu.make_async_copy(rows_v.at[b], accum.at[tgt_all.at[c]],
                                      sema[b]).wait()

                @pl.when(c + RING < cpw)
                def _():
                    issue_load(c + RING, b)
            return carry

        lax.fori_loop(0, cpw // RING, body, 0)
        plsc.subcore_barrier()
        pltpu.sync_copy(accum.at[pl.ds(s_ax * rows_per_sub, rows_per_sub)],
                        part_out.at[c_ax, pl.ds(s_ax * rows_per_sub,
                                                rows_per_sub)])

    return scatter_kernel


def _final_body(x_ref, w_ref, b_ref, p0_ref, p1_ref, o_ref):
    o_ref[...] = (jnp.dot(x_ref[...], w_ref[...],
                          preferred_element_type=jnp.float32)
                  + b_ref[...][None, :] + p0_ref[...] + p1_ref[...])


def _final(x_pad, w, b, p0, p1):
    n_pad = x_pad.shape[0]
    blk = pl.BlockSpec((TC_BLK, F), lambda i: (i, 0))
    return pl.pallas_call(
        _final_body,
        grid=(n_pad // TC_BLK,),
        in_specs=[blk,
                  pl.BlockSpec((F, F), lambda i: (0, 0)),
                  pl.BlockSpec((F,), lambda i: (0,)),
                  blk, blk],
        out_specs=blk,
        out_shape=jax.ShapeDtypeStruct((n_pad, F), jnp.float32),
    )(x_pad, w, b, p0, p1)


def kernel(x, edge_index, pos, W_node, b_node, W_e1, b_e1, W_e2, b_e2,
           W_r1, b_r1, W_r2, b_r2):
    N, Fin = x.shape
    E = edge_index.shape[1]
    mesh = plsc.VectorSubcoreMesh(core_axis_name="c", subcore_axis_name="s")
    nc, ns = mesh.num_cores, mesh.num_subcores
    nw = nc * ns
    cpw = -(-E // (nw * CHUNK))
    cpw = -(-cpw // RING) * RING
    e_pad = nw * CHUNK * cpw
    n_pad = -(-N // TC_BLK) * TC_BLK

    src = edge_index[0].astype(jnp.int32)
    dst = edge_index[1].astype(jnp.int32)
    src3 = jnp.concatenate(
        [src, jnp.full((e_pad - E,), N, jnp.int32)]).reshape(nw, cpw, CHUNK)
    dst3 = jnp.concatenate(
        [dst, jnp.zeros((e_pad - E,), jnp.int32)]).reshape(nw, cpw, CHUNK)
    tgt3 = dst3

    x_pad = jnp.pad(x, ((0, n_pad - N), (0, 0)))
    pos3f = jnp.full((n_pad, 3), 1e6, jnp.float32)
    pos3f = pos3f.at[:N].set(pos).reshape(-1)

    r_hid = W_r1.shape[1]
    r_out = W_r2.shape[1]
    wr1v = jnp.pad(W_r1[0], (0, F - r_hid))
    br1v = jnp.pad(b_r1, (0, F - r_hid))
    wr2p = jnp.zeros((F, F), jnp.float32).at[:r_hid, :r_out].set(W_r2)
    br2v = jnp.pad(b_r2, (0, F - r_out))
    we1bp = jnp.zeros((F, F), jnp.float32).at[:r_out].set(W_e1[Fin:])
    zinit = jnp.zeros((n_pad // ns, F), jnp.float32)

    xe1 = _node_premul(x_pad, W_e1[:Fin])
    g, d2 = _make_gather(nc, ns, cpw, e_pad, n_pad)(xe1, src3, dst3, pos3f)
    ef = _edge_mlp(g, d2.reshape(-1), wr1v, br1v, wr2p, br2v, we1bp,
                   b_e1, W_e2, b_e2)
    part = _make_scatter(nc, ns, cpw, e_pad, n_pad)(ef, tgt3, zinit)
    out_pad = _final(x_pad, W_node, b_node, part[0], part[1])
    return out_pad[:N]


# speedup vs baseline: 3.0184x; 1.0251x over previous
import functools

import jax
import jax.numpy as jnp
from jax import lax
from jax.experimental import pallas as pl
from jax.experimental.pallas import tpu as pltpu
from jax.experimental.pallas import tpu_sc as plsc

F = 128
CHUNK = 128
LANES = 16
TC_BLK = 1024
EDGE_BLK = 1024
NSEG = 4


def _silu(v):
    return v * (1.0 / (1.0 + jnp.exp(-v)))


def _matmul_body(x_ref, w_ref, o_ref):
    o_ref[...] = jnp.dot(x_ref[...], w_ref[...],
                         preferred_element_type=jnp.float32)


def _node_premul(x_pad, w):
    n_pad = x_pad.shape[0]
    return pl.pallas_call(
        _matmul_body,
        grid=(n_pad // TC_BLK,),
        in_specs=[pl.BlockSpec((TC_BLK, F), lambda i: (i, 0)),
                  pl.BlockSpec((F, F), lambda i: (0, 0))],
        out_specs=pl.BlockSpec((TC_BLK, F), lambda i: (i, 0)),
        out_shape=jax.ShapeDtypeStruct((n_pad, F), jnp.float32),
    )(x_pad, w)


RING = 2


def _make_gather(nc, ns, cpw, e_pad, n_pad):
    nw = nc * ns
    mesh = plsc.VectorSubcoreMesh(core_axis_name="c", subcore_axis_name="s")

    @functools.partial(
        pl.kernel,
        out_type=(jax.ShapeDtypeStruct((e_pad, F), jnp.float32),
                  jax.ShapeDtypeStruct((nw, cpw, CHUNK), jnp.float32)),
        mesh=mesh,
        scratch_types=[
            pltpu.VMEM((n_pad * 3,), jnp.float32),
            pltpu.VMEM((cpw, CHUNK), jnp.int32),
            pltpu.VMEM((cpw, CHUNK), jnp.int32),
            pltpu.VMEM((RING, CHUNK, F), jnp.float32),
            pltpu.VMEM((cpw, CHUNK), jnp.float32),
        ] + [pltpu.SemaphoreType.DMA] * (2 * RING),
        compiler_params=pltpu.CompilerParams(needs_layout_passes=False),
    )
    def gather_kernel(xe1, src3, dst3, pos3, g_out, d2_out,
                      pos_v, src_all, dst_all, rows_v, d2_all, *sems):
        semg = sems[:RING]
        semw = sems[RING:]
        wid = lax.axis_index("s") * nc + lax.axis_index("c")
        pltpu.sync_copy(pos3, pos_v)
        pltpu.sync_copy(src3.at[wid], src_all)
        pltpu.sync_copy(dst3.at[wid], dst_all)

        def issue_gather(c, b):
            pltpu.async_copy(xe1.at[src_all.at[c]], rows_v.at[b], semg[b])

        for b in range(RING):
            issue_gather(b, b)

        def body(g, carry):
            for b in range(RING):
                c = g * RING + b
                j = wid * cpw + c
                pltpu.make_async_copy(
                    xe1.at[src_all.at[c]], rows_v.at[b], semg[b]).wait()
                out_slice = g_out.at[pl.ds(j * CHUNK, CHUNK)]
                pltpu.async_copy(rows_v.at[b], out_slice, semw[b])
                for grp in range(CHUNK // LANES):
                    sv = src_all[c, pl.ds(grp * LANES, LANES)] * 3
                    dv = dst_all[c, pl.ds(grp * LANES, LANES)] * 3
                    acc = jnp.zeros((LANES,), jnp.float32)
                    for cdim in range(3):
                        ps = plsc.load_gather(pos_v, [sv + cdim])
                        pd = plsc.load_gather(pos_v, [dv + cdim])
                        d = pd - ps
                        acc = acc + d * d
                    d2_all[c, pl.ds(grp * LANES, LANES)] = acc
                pltpu.make_async_copy(rows_v.at[b], out_slice, semw[b]).wait()

                @pl.when(c + RING < cpw)
                def _():
                    issue_gather(c + RING, b)
            return carry

        lax.fori_loop(0, cpw // RING, body, 0)
        pltpu.sync_copy(d2_all, d2_out.at[wid])

    return gather_kernel


def _edge_mlp_body(g_ref, d2_ref, wr1_ref, br1_ref, wr2_ref, br2_ref,
                   we1b_ref, be1_ref, we2_ref, be2_ref, o_ref):
    g = g_ref[...]
    l2 = d2_ref[...] + 1e-12
    l = jnp.sqrt(l2)
    r_hid = wr2_ref.shape[0]
    r_out = wr2_ref.shape[1]
    wr1 = wr1_ref[...][:r_hid]
    br1 = br1_ref[...][:r_hid]
    hid = _silu(l[:, None] * wr1[None, :] + br1[None, :])
    rad = _silu(jnp.dot(hid, wr2_ref[...],
                        preferred_element_type=jnp.float32)
                + br2_ref[...][None, :r_out])
    radc = be1_ref[...][None, :]
    for k in range(r_out):
        radc = radc + rad[:, k:k + 1] * we1b_ref[k, :][None, :]
    h = _silu(g + radc)
    e = _silu(jnp.dot(h, we2_ref[...], preferred_element_type=jnp.float32)
              + be2_ref[...][None, :])
    cf = jnp.clip(1.0 - (l * 0.1) ** 2, 0.0, 1.0) * (l < 10.0).astype(jnp.float32)
    o_ref[...] = e * cf[:, None]


def _edge_mlp(g, d2, wr1v, br1v, wr2, br2v, we1b, be1, we2, be2):
    e_seg = g.shape[0]
    vec = pl.BlockSpec((F,), lambda i: (0,))
    mat = pl.BlockSpec((F, F), lambda i: (0, 0))
    return pl.pallas_call(
        _edge_mlp_body,
        grid=(e_seg // EDGE_BLK,),
        in_specs=[pl.BlockSpec((EDGE_BLK, F), lambda i: (i, 0)),
                  pl.BlockSpec((EDGE_BLK,), lambda i: (i,)),
                  vec, vec,
                  pl.BlockSpec(wr2.shape, lambda i: (0, 0)), vec,
                  pl.BlockSpec(we1b.shape, lambda i: (0, 0)), vec, mat, vec],
        out_specs=pl.BlockSpec((EDGE_BLK, F), lambda i: (i, 0)),
        out_shape=jax.ShapeDtypeStruct((e_seg, F), jnp.float32),
    )(g, d2, wr1v, br1v, wr2, br2v, we1b, be1, we2, be2)


def _make_scatter(nc, ns, cpw, e_pad, n_pad):
    mesh = plsc.VectorSubcoreMesh(core_axis_name="c", subcore_axis_name="s")
    rows_per_sub = n_pad // ns

    @functools.partial(
        pl.kernel,
        out_type=jax.ShapeDtypeStruct((nc, n_pad, F), jnp.float32),
        mesh=mesh,
        scratch_types=[
            pltpu.VMEM((NSEG, cpw, CHUNK), jnp.int32),
            pltpu.VMEM((RING, CHUNK, F), jnp.float32),
            pltpu.VMEM_SHARED((n_pad, F), jnp.float32),
        ] + [pltpu.SemaphoreType.DMA] * (2 * RING),
        compiler_params=pltpu.CompilerParams(needs_layout_passes=False),
    )
    def scatter_kernel(*args):
        efs = args[:NSEG]
        tgt4, zinit, part_out, tgt_all, rows_v, accum = args[NSEG:NSEG + 6]
        sems = args[NSEG + 6:]
        seml = sems[:RING]
        sema = sems[RING:]
        c_ax = lax.axis_index("c")
        s_ax = lax.axis_index("s")
        wid = s_ax * nc + c_ax
        for s in range(NSEG):
            pltpu.sync_copy(tgt4.at[s, wid], tgt_all.at[s])
        pltpu.sync_copy(zinit, accum.at[pl.ds(s_ax * rows_per_sub,
                                              rows_per_sub)])
        plsc.subcore_barrier()

        for s in range(NSEG):
            ef = efs[s]

            def issue_load(c, b):
                j = wid * cpw + c
                pltpu.async_copy(ef.at[pl.ds(j * CHUNK, CHUNK)],
                                 rows_v.at[b], seml[b])

            for b in range(RING):
                issue_load(b, b)

            def body(g, carry):
                for b in range(RING):
                    c = g * RING + b
                    j = wid * cpw + c
                    idx = tgt_all.at[s, c]
                    pltpu.make_async_copy(ef.at[pl.ds(j * CHUNK, CHUNK)],
                                          rows_v.at[b], seml[b]).wait()
                    pltpu.async_copy(rows_v.at[b], accum.at[idx],
                                     sema[b], add=True)
                    pltpu.make_async_copy(rows_v.at[b], accum.at[idx],
                                          sema[b]).wait()

                    @pl.when(c + RING < cpw)
                    def _():
                        issue_load(c + RING, b)
                return carry

            lax.fori_loop(0, cpw // RING, body, 0)
        plsc.subcore_barrier()
        pltpu.sync_copy(accum.at[pl.ds(s_ax * rows_per_sub, rows_per_sub)],
                        part_out.at[c_ax, pl.ds(s_ax * rows_per_sub,
                                                rows_per_sub)])

    return scatter_kernel


def _final_body(x_ref, w_ref, b_ref, p0_ref, p1_ref, o_ref):
    o_ref[...] = (jnp.dot(x_ref[...], w_ref[...],
                          preferred_element_type=jnp.float32)
                  + b_ref[...][None, :] + p0_ref[...] + p1_ref[...])


def _final(x_pad, w, b, p0, p1):
    n_pad = x_pad.shape[0]
    blk = pl.BlockSpec((TC_BLK, F), lambda i: (i, 0))
    return pl.pallas_call(
        _final_body,
        grid=(n_pad // TC_BLK,),
        in_specs=[blk,
                  pl.BlockSpec((F, F), lambda i: (0, 0)),
                  pl.BlockSpec((F,), lambda i: (0,)),
                  blk, blk],
        out_specs=blk,
        out_shape=jax.ShapeDtypeStruct((n_pad, F), jnp.float32),
    )(x_pad, w, b, p0, p1)


def kernel(x, edge_index, pos, W_node, b_node, W_e1, b_e1, W_e2, b_e2,
           W_r1, b_r1, W_r2, b_r2):
    N, Fin = x.shape
    E = edge_index.shape[1]
    mesh = plsc.VectorSubcoreMesh(core_axis_name="c", subcore_axis_name="s")
    nc, ns = mesh.num_cores, mesh.num_subcores
    nw = nc * ns
    cps = -(-E // (NSEG * nw * CHUNK))
    cps = -(-cps // RING) * RING
    e_seg = nw * CHUNK * cps
    e_pad = NSEG * e_seg
    n_pad = -(-N // TC_BLK) * TC_BLK

    src = edge_index[0].astype(jnp.int32)
    dst = edge_index[1].astype(jnp.int32)
    src4 = jnp.concatenate(
        [src, jnp.full((e_pad - E,), N, jnp.int32)]).reshape(
            NSEG, nw, cps, CHUNK)
    dst4 = jnp.concatenate(
        [dst, jnp.zeros((e_pad - E,), jnp.int32)]).reshape(
            NSEG, nw, cps, CHUNK)

    x_pad = jnp.pad(x, ((0, n_pad - N), (0, 0)))
    pos3f = jnp.full((n_pad, 3), 1e6, jnp.float32)
    pos3f = pos3f.at[:N].set(pos).reshape(-1)

    r_hid = W_r1.shape[1]
    r_out = W_r2.shape[1]
    wr1v = jnp.pad(W_r1[0], (0, F - r_hid))
    br1v = jnp.pad(b_r1, (0, F - r_hid))
    br2v = jnp.pad(b_r2, (0, F - r_out))
    we1b = W_e1[Fin:]
    zinit = jnp.zeros((n_pad // ns, F), jnp.float32)

    xe1 = _node_premul(x_pad, W_e1[:Fin])
    gather_fn = _make_gather(nc, ns, cps, e_seg, n_pad)
    efs = []
    for s in range(NSEG):
        g, d2 = gather_fn(xe1, src4[s], dst4[s], pos3f)
        efs.append(_edge_mlp(g, d2.reshape(-1), wr1v, br1v, W_r2, br2v,
                             we1b, b_e1, W_e2, b_e2))
    part = _make_scatter(nc, ns, cps, e_seg, n_pad)(*efs, dst4, zinit)
    out_pad = _final(x_pad, W_node, b_node, part[0], part[1])
    return out_pad[:N]
